# baseline (speedup 1.0000x reference)
import jax
import jax.numpy as jnp
from jax.experimental import pallas as pl
from jax.experimental.pallas import tpu as pltpu

B = 16384
D = 2176
HID = 64
BM = 1024
NS = 2
NBS = B // NS // BM


def _mlp_body(*refs):
    x_refs = refs[:NS]
    wh_ref, bh_ref, wo_ref, bo_ref, o_ref = refs[NS:]
    wh = wh_ref[...]
    wo = wo_ref[...]
    bh = bh_ref[...]
    bo = bo_ref[...]
    for s in range(NS):
        x16 = x_refs[s][...].astype(jnp.bfloat16)
        h = jnp.dot(x16, wh, preferred_element_type=jnp.float32) + bh
        z = jnp.dot(h, wo, preferred_element_type=jnp.float32) + bo
        o_ref[s] = jax.nn.sigmoid(z)


def kernel(inputs, W_h, b_h, W_o, b_o):
    W_h = W_h.astype(jnp.bfloat16)
    bh2 = b_h.reshape(1, HID)
    bo2 = b_o.reshape(1, 1)
    in_specs = [
        pl.BlockSpec((BM, D), lambda i, s=s: (i + s * NBS, 0)) for s in range(NS)
    ]
    in_specs += [
        pl.BlockSpec((D, HID), lambda i: (0, 0)),
        pl.BlockSpec((1, HID), lambda i: (0, 0)),
        pl.BlockSpec((HID, 1), lambda i: (0, 0)),
        pl.BlockSpec((1, 1), lambda i: (0, 0)),
    ]
    out = pl.pallas_call(
        _mlp_body,
        grid=(NBS,),
        in_specs=in_specs,
        out_specs=pl.BlockSpec((NS, BM, 1), lambda i: (0, i, 0)),
        out_shape=jax.ShapeDtypeStruct((NS, B // NS, 1), jnp.float32),
        compiler_params=pltpu.CompilerParams(
            dimension_semantics=("arbitrary",),
        ),
    )(*([inputs] * NS), W_h, bh2, W_o, bo2)
    return out.reshape(B, 1)


# speedup vs baseline: 1.0080x; 1.0080x over previous
import jax
import jax.numpy as jnp
from jax.experimental import pallas as pl
from jax.experimental.pallas import tpu as pltpu

B = 16384
D = 2176
HID = 64
CM = 512
NCHUNK = B // CM
NSLOT = 4


def _mlp_body(x_hbm, wh_ref, bh_ref, wo_ref, bo_ref, o_ref, x_vmem, sems):
    wh = wh_ref[...]
    bh = bh_ref[...]
    wo = wo_ref[...]
    bo = bo_ref[...]

    def copy(i, slot):
        return pltpu.make_async_copy(
            x_hbm.at[pl.ds(i * CM, CM), :],
            x_vmem.at[slot],
            sems.at[slot],
        )

    for s in range(NSLOT):
        copy(s, s).start()

    for i in range(NCHUNK):
        slot = i % NSLOT
        copy(i, slot).wait()
        x16 = x_vmem[slot].astype(jnp.bfloat16)
        h = jnp.dot(x16, wh, preferred_element_type=jnp.float32) + bh
        z = jnp.dot(h, wo, preferred_element_type=jnp.float32) + bo
        o_ref[pl.ds(i * CM, CM), :] = jax.nn.sigmoid(z)
        if i + NSLOT < NCHUNK:
            copy(i + NSLOT, slot).start()


def kernel(inputs, W_h, b_h, W_o, b_o):
    W_h = W_h.astype(jnp.bfloat16)
    bh2 = b_h.reshape(1, HID)
    bo2 = b_o.reshape(1, 1)
    out = pl.pallas_call(
        _mlp_body,
        in_specs=[
            pl.BlockSpec(memory_space=pltpu.HBM),
            pl.BlockSpec(memory_space=pltpu.VMEM),
            pl.BlockSpec(memory_space=pltpu.VMEM),
            pl.BlockSpec(memory_space=pltpu.VMEM),
            pl.BlockSpec(memory_space=pltpu.VMEM),
        ],
        out_specs=pl.BlockSpec(memory_space=pltpu.VMEM),
        out_shape=jax.ShapeDtypeStruct((B, 1), jnp.float32),
        scratch_shapes=[
            pltpu.VMEM((NSLOT, CM, D), jnp.float32),
            pltpu.SemaphoreType.DMA((NSLOT,)),
        ],
    )(inputs, W_h, bh2, W_o, bo2)
    return out


# speedup vs baseline: 1.0529x; 1.0445x over previous
import jax
import jax.numpy as jnp
from jax.experimental import pallas as pl
from jax.experimental.pallas import tpu as pltpu

B = 16384
D = 2176
HID = 64
BM = 1024


def _mlp_body(x_ref, wh_ref, bh_ref, wo_ref, bo_ref, o_ref):
    w = jnp.dot(wh_ref[...], wo_ref[...], preferred_element_type=jnp.float32)
    c = jnp.dot(bh_ref[...], wo_ref[...], preferred_element_type=jnp.float32)
    z = jnp.dot(x_ref[...], w, preferred_element_type=jnp.float32)
    o_ref[...] = jax.nn.sigmoid(z + (c + bo_ref[...]))


def kernel(inputs, W_h, b_h, W_o, b_o):
    bh2 = b_h.reshape(1, HID)
    bo2 = b_o.reshape(1, 1)
    out = pl.pallas_call(
        _mlp_body,
        grid=(B // BM,),
        in_specs=[
            pl.BlockSpec((BM, D), lambda i: (i, 0)),
            pl.BlockSpec((D, HID), lambda i: (0, 0)),
            pl.BlockSpec((1, HID), lambda i: (0, 0)),
            pl.BlockSpec((HID, 1), lambda i: (0, 0)),
            pl.BlockSpec((1, 1), lambda i: (0, 0)),
        ],
        out_specs=pl.BlockSpec((BM, 1), lambda i: (i, 0)),
        out_shape=jax.ShapeDtypeStruct((B, 1), jnp.float32),
        compiler_params=pltpu.CompilerParams(
            dimension_semantics=("arbitrary",),
        ),
    )(inputs, W_h, bh2, W_o, bo2)
    return out


# speedup vs baseline: 1.2785x; 1.2143x over previous
import jax
import jax.numpy as jnp
from jax.experimental import pallas as pl
from jax.experimental.pallas import tpu as pltpu

B = 16384
D = 2176
HID = 64
BM = 1024


def _mlp_body(x_ref, wh_ref, bh_ref, wo_ref, bo_ref, o_ref):
    wt = jax.lax.dot_general(
        wo_ref[...], wh_ref[...], (((0,), (1,)), ((), ())),
        preferred_element_type=jnp.float32,
    )
    c = jnp.dot(bh_ref[...], wo_ref[...], preferred_element_type=jnp.float32)
    z = jax.lax.dot_general(
        wt, x_ref[...], (((1,), (1,)), ((), ())),
        preferred_element_type=jnp.float32,
    )
    o_ref[...] = jax.nn.sigmoid(z + (c + bo_ref[...])).reshape(1, 1, BM)


def kernel(inputs, W_h, b_h, W_o, b_o):
    bh2 = b_h.reshape(1, HID)
    bo2 = b_o.reshape(1, 1)
    out = pl.pallas_call(
        _mlp_body,
        grid=(B // BM,),
        in_specs=[
            pl.BlockSpec((BM, D), lambda i: (i, 0)),
            pl.BlockSpec((D, HID), lambda i: (0, 0)),
            pl.BlockSpec((1, HID), lambda i: (0, 0)),
            pl.BlockSpec((HID, 1), lambda i: (0, 0)),
            pl.BlockSpec((1, 1), lambda i: (0, 0)),
        ],
        out_specs=pl.BlockSpec((1, 1, BM), lambda i: (i, 0, 0)),
        out_shape=jax.ShapeDtypeStruct((B // BM, 1, BM), jnp.float32),
        compiler_params=pltpu.CompilerParams(
            dimension_semantics=("arbitrary",),
        ),
    )(inputs, W_h, bh2, W_o, bo2)
    return out.reshape(B, 1)
